# baseline (speedup 1.0000x reference)
import functools

import jax
import jax.numpy as jnp
from jax.experimental import pallas as pl
from jax.experimental.pallas import tpu as pltpu
from jax.experimental.pallas import tpu_sc as plsc

N = 10000
E = 160000
H = 128
NFD = 4 * H
NGRAPH = 64
VOCAB = 100

BN = 1000
BE = 1600
GW = 128
SBLK = 128
NSB = E // SBLK
NPS = 640


_vmesh = functools.partial(
    plsc.VectorSubcoreMesh, core_axis_name="c", subcore_axis_name="s")


def _embed_body(zf_ref, emb_ref, x_ref):
    lane = jax.lax.broadcasted_iota(jnp.int32, (1, 128), 1).astype(jnp.float32)
    oh = (zf_ref[...] == lane).astype(jnp.float32)
    x_ref[...] = jnp.dot(oh, emb_ref[...],
                         preferred_element_type=jnp.float32)


def _embed(zf, emb_pad):
    return pl.pallas_call(
        _embed_body,
        grid=(N // BN,),
        in_specs=[pl.BlockSpec((BN, 1), lambda i: (i, 0)),
                  pl.BlockSpec((128, 128), lambda i: (0, 0))],
        out_specs=pl.BlockSpec((BN, H), lambda i: (i, 0)),
        out_shape=jax.ShapeDtypeStruct((N, H), jnp.float32),
    )(zf, emb_pad)


def _bf16_bits(x):
    u = jax.lax.bitcast_convert_type(x, jnp.int32)
    r = u + 0x7FFF + jnp.bitwise_and(jnp.right_shift(u, 16), 1)
    return jnp.bitwise_and(jnp.right_shift(r, 16), 0xFFFF)


def _nodeproj_body(x_ref, v_ref, wab_ref, a_ref, b_ref):
    nf = jnp.concatenate([x_ref[...], v_ref[...]], axis=1)
    ab = jnp.dot(nf, wab_ref[...], preferred_element_type=jnp.float32)
    v = v_ref[...]
    comps = [jnp.mean(v[:, i * H:(i + 1) * H], axis=1, keepdims=True)
             for i in range(3)]
    ppw = _bf16_bits(jnp.concatenate(
        comps + [jnp.zeros((v.shape[0], 125), jnp.float32)], axis=1))
    ha = NFD // 2
    aw = jnp.bitwise_or(_bf16_bits(ab[:, 0 * ha:1 * ha]),
                        jnp.left_shift(_bf16_bits(ab[:, 1 * ha:2 * ha]), 16))
    bw = jnp.bitwise_or(_bf16_bits(ab[:, 2 * ha:3 * ha]),
                        jnp.left_shift(_bf16_bits(ab[:, 3 * ha:4 * ha]), 16))
    a_ref[...] = jnp.concatenate([aw, ppw], axis=1)
    b_ref[...] = jnp.concatenate([bw, ppw], axis=1)


def _nodeproj(x, v_flat, wab):
    return pl.pallas_call(
        _nodeproj_body,
        grid=(N // BN,),
        in_specs=[pl.BlockSpec((BN, H), lambda i: (i, 0)),
                  pl.BlockSpec((BN, 3 * H), lambda i: (i, 0)),
                  pl.BlockSpec((NFD, 2 * NFD), lambda i: (0, 0))],
        out_specs=[pl.BlockSpec((BN, 384), lambda i: (i, 0))] * 2,
        out_shape=[jax.ShapeDtypeStruct((N, 384), jnp.int32)] * 2,
    )(x, v_flat, wab)


def _sc_gather(table, idx2, window):
    d = table.shape[1]
    e = idx2.shape[1]

    @functools.partial(
        pl.kernel,
        out_type=jax.ShapeDtypeStruct((e, d), table.dtype),
        mesh=_vmesh())
    def k(tab_hbm, i_hbm, o_hbm):
        def body(i_vmem, o_vmem):
            pltpu.sync_copy(tab_hbm.at[i_vmem.at[0]], o_vmem)

        pltpu.emit_pipeline(
            body,
            grid=(e // window,),
            in_specs=[pl.BlockSpec((1, window), lambda i: (0, i))],
            out_specs=[pl.BlockSpec((window, d), lambda i: (i, 0))],
            core_axis_name=("c", "s"),
            dimension_semantics=(pltpu.PARALLEL,),
        )(i_hbm, o_hbm)

    return k(table, idx2)


def _unpack_lo(w):
    return jax.lax.bitcast_convert_type(jnp.left_shift(w, 16), jnp.float32)


def _unpack_hi(w):
    return jax.lax.bitcast_convert_type(
        jnp.bitwise_and(w, jnp.int32(-65536)), jnp.float32)


def _edge_body(ga_ref, gb_ref, w2_ref,
               cst_ref, m0_ref, m1_ref, m2_ref, m3_ref):
    wd = cst_ref[0:1, :]
    b1 = cst_ref[1:2, :]
    b2 = cst_ref[2:3, :]
    ha = NFD // 2
    gaw = ga_ref[...]
    gbw = gb_ref[...]
    dr = _unpack_lo(gaw[:, ha:]) - _unpack_lo(gbw[:, ha:])
    dist = jnp.sqrt(jnp.sum(dr * dr, axis=1, keepdims=True))
    ga = jnp.concatenate([_unpack_lo(gaw[:, :ha]), _unpack_hi(gaw[:, :ha])],
                         axis=1)
    gb = jnp.concatenate([_unpack_lo(gbw[:, :ha]), _unpack_hi(gbw[:, :ha])],
                         axis=1)
    u = ga + gb + dist * wd + b1
    h1 = u * jax.nn.sigmoid(u)
    m = jnp.dot(h1.astype(jnp.bfloat16), w2_ref[...],
                preferred_element_type=jnp.float32) + b2
    m = m * jax.nn.sigmoid(m)
    m0_ref[...] = m[:, 0 * H:1 * H]
    m1_ref[...] = m[:, 1 * H:2 * H]
    m2_ref[...] = m[:, 2 * H:3 * H]
    m3_ref[...] = m[:, 3 * H:4 * H]


def _edge_mlp(ga, gb, w2, cst):
    e = ga.shape[0]
    outs = [jax.ShapeDtypeStruct((e, H), jnp.float32)] * 4
    return pl.pallas_call(
        _edge_body,
        grid=(e // BE,),
        in_specs=[pl.BlockSpec((BE, 384), lambda i: (i, 0)),
                  pl.BlockSpec((BE, 384), lambda i: (i, 0)),
                  pl.BlockSpec((NFD, NFD), lambda i: (0, 0)),
                  pl.BlockSpec((8, NFD), lambda i: (0, 0))],
        out_specs=[pl.BlockSpec((BE, H), lambda i: (i, 0))] * 4,
        out_shape=outs,
    )(ga, gb, w2, cst)


def _sc_scatter_add(m0, m1, m2, m3, idxb, zrows):
    out_t = tuple(jax.ShapeDtypeStruct((N, H), jnp.float32) for _ in range(4))
    DB = SBLK
    ND = idxb.shape[0]
    NPB = (ND + 15) // 16

    @functools.partial(
        pl.kernel,
        out_type=out_t,
        mesh=_vmesh(),
        scratch_types=[pltpu.VMEM_SHARED((N, H), jnp.float32),
                       pltpu.VMEM((2 * DB, H), jnp.float32),
                       pltpu.VMEM((2, 1, SBLK), jnp.int32),
                       pltpu.SemaphoreType.DMA,
                       pltpu.SemaphoreType.DMA,
                       pltpu.SemaphoreType.DMA,
                       pltpu.SemaphoreType.DMA])
    def k(m0h, m1h, m2h, m3h, ih, zh, o0, o1, o2, o3,
          acc, mbuf, ibuf, si0, si1, sm0, sm1):
        c = jax.lax.axis_index("c")
        s = jax.lax.axis_index("s")
        r0 = s * NPS
        isems = (si0, si1)
        msems = (sm0, sm1)

        def do_chunk(mh, oh):
            @pl.when(s < 15)
            def _():
                pltpu.sync_copy(zh, acc.at[pl.ds(r0, NPS)])

            @pl.when(s == 15)
            def _():
                pltpu.sync_copy(zh.at[pl.ds(0, N - 15 * NPS)],
                                acc.at[pl.ds(15 * NPS, N - 15 * NPS)])

            plsc.subcore_barrier()

            def issue(b, par):
                blk = b * 16 + s

                @pl.when(blk < ND)
                def _():
                    pltpu.make_async_copy(
                        ih.at[pl.ds(blk, 1)], ibuf.at[par],
                        isems[par]).start()
                    pltpu.make_async_copy(
                        mh.at[pl.ds(blk * DB, DB)],
                        mbuf.at[pl.ds(par * DB, DB)], msems[par]).start()

            def consume(b, par):
                blk = b * 16 + s

                @pl.when(blk < ND)
                def _():
                    pltpu.make_async_copy(
                        ih.at[pl.ds(blk, 1)], ibuf.at[par],
                        isems[par]).wait()
                    pltpu.make_async_copy(
                        mh.at[pl.ds(blk * DB, DB)],
                        mbuf.at[pl.ds(par * DB, DB)], msems[par]).wait()
                    pltpu.sync_copy(
                        mbuf.at[pl.ds(par * DB, DB)],
                        acc.at[ibuf.at[par, 0]], add=True)

            issue(0, 0)

            @pl.loop(0, (NPB + 1) // 2)
            def _(i):
                b0 = 2 * i
                issue(b0 + 1, 1)
                consume(b0, 0)
                issue(b0 + 2, 0)
                consume(b0 + 1, 1)

            plsc.subcore_barrier()

            @pl.when(s < 15)
            def _():
                pltpu.sync_copy(acc.at[pl.ds(r0, NPS)], oh.at[pl.ds(r0, NPS)])

            @pl.when(s == 15)
            def _():
                pltpu.sync_copy(acc.at[pl.ds(15 * NPS, N - 15 * NPS)],
                                oh.at[pl.ds(15 * NPS, N - 15 * NPS)])

            plsc.subcore_barrier()

        @pl.when(c == 0)
        def _():
            do_chunk(m0h, o0)
            do_chunk(m1h, o1)

        @pl.when(c == 1)
        def _():
            do_chunk(m2h, o2)
            do_chunk(m3h, o3)

    return k(m0, m1, m2, m3, idxb, zrows)


def _nodeupd_body(x_ref, v_ref, *rest):
    aggs = rest[:20]
    sw1_ref, sw2_ref, vm_ref, cst_ref, xo_ref, vo_ref = rest[20:]
    sb1 = cst_ref[0:1, :]
    sb2 = cst_ref[1:2, :]
    x = x_ref[...]
    summed = []
    for f in range(4):
        t = aggs[f][...]
        for h in range(1, 5):
            t = t + aggs[4 * h + f][...]
        summed.append(t)
    xu = jnp.concatenate([x, summed[0]], axis=1)
    t = jnp.dot(xu, sw1_ref[...], preferred_element_type=jnp.float32) + sb1
    t = t * jax.nn.sigmoid(t)
    xo_ref[...] = x + jnp.dot(t, sw2_ref[...],
                              preferred_element_type=jnp.float32) + sb2
    v = v_ref[...]
    comps = []
    for i in range(3):
        vi = v[:, i * H:(i + 1) * H]
        comps.append(vi + summed[1 + i] +
                     jnp.dot(vi, vm_ref[...],
                             preferred_element_type=jnp.float32))
    vo_ref[...] = jnp.concatenate(comps, axis=1)


def _nodeupd(x, v_flat, aggsets, sw1, sw2, vm, cst):
    flat = [a for s in aggsets for a in s]
    return pl.pallas_call(
        _nodeupd_body,
        grid=(N // BN,),
        in_specs=[pl.BlockSpec((BN, H), lambda i: (i, 0)),
                  pl.BlockSpec((BN, 3 * H), lambda i: (i, 0))] +
                 [pl.BlockSpec((BN, H), lambda i: (i, 0))] * 20 +
                 [pl.BlockSpec((2 * H, H), lambda i: (0, 0)),
                  pl.BlockSpec((H, H), lambda i: (0, 0)),
                  pl.BlockSpec((H, H), lambda i: (0, 0)),
                  pl.BlockSpec((8, H), lambda i: (0, 0))],
        out_specs=[pl.BlockSpec((BN, H), lambda i: (i, 0)),
                   pl.BlockSpec((BN, 3 * H), lambda i: (i, 0))],
        out_shape=[jax.ShapeDtypeStruct((N, H), jnp.float32),
                   jax.ShapeDtypeStruct((N, 3 * H), jnp.float32)],
    )(x, v_flat, *flat, sw1, sw2, vm, cst)


def _readout_body(v_ref, bf_ref, wf_ref, dip_ref, mol_ref):
    k = pl.program_id(0)

    @pl.when(k == 0)
    def _():
        mol_ref[...] = jnp.zeros_like(mol_ref)

    seg = jax.lax.broadcasted_iota(jnp.int32, (1, NGRAPH), 1).astype(jnp.float32)
    oh = (bf_ref[...] == seg).astype(jnp.float32)
    mol_ref[...] += jax.lax.dot_general(
        oh, v_ref[...], (((0,), (0,)), ((), ())),
        preferred_element_type=jnp.float32)

    @pl.when(k == N // BN - 1)
    def _():
        mol = mol_ref[...]
        wf = wf_ref[0:1, :]
        lane = jax.lax.broadcasted_iota(jnp.int32, (1, 128), 1)
        acc = jnp.zeros((NGRAPH, 128), jnp.float32)
        for i in range(3):
            ti = jnp.sum(mol[:, i * H:(i + 1) * H] * wf, axis=1,
                         keepdims=True)
            acc += ti * (lane == i).astype(jnp.float32)
        dip_ref[...] = acc


def _readout(v_flat, bf, wfr):
    return pl.pallas_call(
        _readout_body,
        grid=(N // BN,),
        in_specs=[pl.BlockSpec((BN, 3 * H), lambda i: (i, 0)),
                  pl.BlockSpec((BN, 1), lambda i: (i, 0)),
                  pl.BlockSpec((8, H), lambda i: (0, 0))],
        out_specs=pl.BlockSpec((NGRAPH, 128), lambda i: (0, 0)),
        out_shape=jax.ShapeDtypeStruct((NGRAPH, 128), jnp.float32),
        scratch_shapes=[pltpu.VMEM((NGRAPH, 3 * H), jnp.float32)],
    )(v_flat, bf, wfr)


def kernel(params, pos, z, edge_index, batch):
    row = edge_index[0].astype(jnp.int32)
    col = edge_index[1].astype(jnp.int32)
    NCH = 5
    EC = E // NCH
    rows2 = [row[h * EC:(h + 1) * EC].reshape(1, EC) for h in range(NCH)]
    cols2 = [col[h * EC:(h + 1) * EC].reshape(1, EC) for h in range(NCH)]
    rowbs = [row[h * EC:(h + 1) * EC].reshape(EC // SBLK, SBLK)
             for h in range(NCH)]

    emb_pad = jnp.zeros((128, H), jnp.float32).at[:VOCAB].set(params['emb'])
    zf = z.astype(jnp.float32).reshape(N, 1)
    x = _embed(zf, emb_pad)

    v_flat = jnp.broadcast_to(pos[:, :, None], (N, 3, H)).astype(
        jnp.float32).reshape(N, 3 * H)

    zrows = jnp.zeros((NPS, H), jnp.float32)
    outs = []
    for lp in params['layers']:
        wab = jnp.concatenate([lp['mw1'][:NFD], lp['mw1'][NFD:2 * NFD]],
                              axis=1)
        cst_e = jnp.zeros((8, NFD), jnp.float32)
        cst_e = cst_e.at[0].set(lp['mw1'][2 * NFD])
        cst_e = cst_e.at[1].set(lp['mb1'])
        cst_e = cst_e.at[2].set(lp['mb2'])
        cst_n = jnp.zeros((8, H), jnp.float32)
        cst_n = cst_n.at[0].set(lp['sb1'])
        cst_n = cst_n.at[1].set(lp['sb2'])

        a, b = _nodeproj(x, v_flat, wab)
        w2b = lp['mw2'].astype(jnp.bfloat16)
        aggs = []
        for h in range(NCH):
            ga = _sc_gather(a, rows2[h], GW)
            gb = _sc_gather(b, cols2[h], GW)
            m0, m1, m2, m3 = _edge_mlp(ga, gb, w2b, cst_e)
            aggs.append(_sc_scatter_add(m0, m1, m2, m3, rowbs[h], zrows))
        x, v_flat = _nodeupd(x, v_flat, aggs,
                             lp['sw1'], lp['sw2'], lp['vm'], cst_n)
        outs.append(x)
        outs.append(v_flat.reshape(N, 3, H))

    bf = batch.astype(jnp.float32).reshape(N, 1)
    wfr = jnp.zeros((8, H), jnp.float32).at[0].set(params['wf'][:, 0])
    dipfull = _readout(v_flat, bf, wfr)
    dip = dipfull[:NGRAPH, :3]
    return (dip, *outs)


# speedup vs baseline: 1.1876x; 1.1876x over previous
import functools

import jax
import jax.numpy as jnp
from jax.experimental import pallas as pl
from jax.experimental.pallas import tpu as pltpu
from jax.experimental.pallas import tpu_sc as plsc

N = 10000
E = 160000
H = 128
NFD = 4 * H
NGRAPH = 64
VOCAB = 100

BN = 1000
BE = 1600
GW = 128
SBLK = 128
NSB = E // SBLK
NPS = 640


_vmesh = functools.partial(
    plsc.VectorSubcoreMesh, core_axis_name="c", subcore_axis_name="s")


def _embed_body(zf_ref, emb_ref, x_ref):
    lane = jax.lax.broadcasted_iota(jnp.int32, (1, 128), 1).astype(jnp.float32)
    oh = (zf_ref[...] == lane).astype(jnp.float32)
    x_ref[...] = jnp.dot(oh, emb_ref[...],
                         preferred_element_type=jnp.float32)


def _embed(zf, emb_pad):
    return pl.pallas_call(
        _embed_body,
        grid=(N // BN,),
        in_specs=[pl.BlockSpec((BN, 1), lambda i: (i, 0)),
                  pl.BlockSpec((128, 128), lambda i: (0, 0))],
        out_specs=pl.BlockSpec((BN, H), lambda i: (i, 0)),
        out_shape=jax.ShapeDtypeStruct((N, H), jnp.float32),
    )(zf, emb_pad)


def _bf16_bits(x):
    u = jax.lax.bitcast_convert_type(x, jnp.int32)
    r = u + 0x7FFF + jnp.bitwise_and(jnp.right_shift(u, 16), 1)
    return jnp.bitwise_and(jnp.right_shift(r, 16), 0xFFFF)


def _nodeproj_body(x_ref, v_ref, wab_ref, a_ref, b_ref):
    nf = jnp.concatenate([x_ref[...], v_ref[...]], axis=1)
    ab = jnp.dot(nf, wab_ref[...], preferred_element_type=jnp.float32)
    v = v_ref[...]
    comps = [jnp.mean(v[:, i * H:(i + 1) * H], axis=1, keepdims=True)
             for i in range(3)]
    ppw = _bf16_bits(jnp.concatenate(
        comps + [jnp.zeros((v.shape[0], 125), jnp.float32)], axis=1))
    ha = NFD // 2
    aw = jnp.bitwise_or(_bf16_bits(ab[:, 0 * ha:1 * ha]),
                        jnp.left_shift(_bf16_bits(ab[:, 1 * ha:2 * ha]), 16))
    bw = jnp.bitwise_or(_bf16_bits(ab[:, 2 * ha:3 * ha]),
                        jnp.left_shift(_bf16_bits(ab[:, 3 * ha:4 * ha]), 16))
    a_ref[...] = jnp.concatenate([aw, ppw], axis=1)
    b_ref[...] = jnp.concatenate([bw, ppw], axis=1)


def _nodeproj(x, v_flat, wab):
    return pl.pallas_call(
        _nodeproj_body,
        grid=(N // BN,),
        in_specs=[pl.BlockSpec((BN, H), lambda i: (i, 0)),
                  pl.BlockSpec((BN, 3 * H), lambda i: (i, 0)),
                  pl.BlockSpec((NFD, 2 * NFD), lambda i: (0, 0))],
        out_specs=[pl.BlockSpec((BN, 384), lambda i: (i, 0))] * 2,
        out_shape=[jax.ShapeDtypeStruct((N, 384), jnp.int32)] * 2,
    )(x, v_flat, wab)


def _sc_gather(table, idx2, window):
    d = table.shape[1]
    e = idx2.shape[1]

    @functools.partial(
        pl.kernel,
        out_type=jax.ShapeDtypeStruct((e, d), table.dtype),
        mesh=_vmesh())
    def k(tab_hbm, i_hbm, o_hbm):
        def body(i_vmem, o_vmem):
            pltpu.sync_copy(tab_hbm.at[i_vmem.at[0]], o_vmem)

        pltpu.emit_pipeline(
            body,
            grid=(e // window,),
            in_specs=[pl.BlockSpec((1, window), lambda i: (0, i))],
            out_specs=[pl.BlockSpec((window, d), lambda i: (i, 0))],
            core_axis_name=("c", "s"),
            dimension_semantics=(pltpu.PARALLEL,),
        )(i_hbm, o_hbm)

    return k(table, idx2)


def _unpack_lo(w):
    return jax.lax.bitcast_convert_type(jnp.left_shift(w, 16), jnp.float32)


def _unpack_hi(w):
    return jax.lax.bitcast_convert_type(
        jnp.bitwise_and(w, jnp.int32(-65536)), jnp.float32)


def _edge_body(ga_ref, gb_ref, w2_ref,
               cst_ref, m0_ref, m1_ref, m2_ref, m3_ref):
    wd = cst_ref[0:1, :]
    b1 = cst_ref[1:2, :]
    b2 = cst_ref[2:3, :]
    ha = NFD // 2
    gaw = ga_ref[...]
    gbw = gb_ref[...]
    dr = _unpack_lo(gaw[:, ha:]) - _unpack_lo(gbw[:, ha:])
    dist = jnp.sqrt(jnp.sum(dr * dr, axis=1, keepdims=True))
    ga = jnp.concatenate([_unpack_lo(gaw[:, :ha]), _unpack_hi(gaw[:, :ha])],
                         axis=1)
    gb = jnp.concatenate([_unpack_lo(gbw[:, :ha]), _unpack_hi(gbw[:, :ha])],
                         axis=1)
    u = ga + gb + dist * wd + b1
    h1 = u * jax.nn.sigmoid(u)
    m = jnp.dot(h1.astype(jnp.bfloat16), w2_ref[...],
                preferred_element_type=jnp.float32) + b2
    m = m * jax.nn.sigmoid(m)
    m0_ref[...] = m[:, 0 * H:1 * H]
    m1_ref[...] = m[:, 1 * H:2 * H]
    m2_ref[...] = m[:, 2 * H:3 * H]
    m3_ref[...] = m[:, 3 * H:4 * H]


def _edge_mlp(ga, gb, w2, cst):
    e = ga.shape[0]
    outs = [jax.ShapeDtypeStruct((e, H), jnp.float32)] * 4
    return pl.pallas_call(
        _edge_body,
        grid=(e // BE,),
        in_specs=[pl.BlockSpec((BE, 384), lambda i: (i, 0)),
                  pl.BlockSpec((BE, 384), lambda i: (i, 0)),
                  pl.BlockSpec((NFD, NFD), lambda i: (0, 0)),
                  pl.BlockSpec((8, NFD), lambda i: (0, 0))],
        out_specs=[pl.BlockSpec((BE, H), lambda i: (i, 0))] * 4,
        out_shape=outs,
    )(ga, gb, w2, cst)


def _sc_scatter_add(m0, m1, m2, m3, idxb, zrows):
    out_t = tuple(jax.ShapeDtypeStruct((N, H), jnp.float32) for _ in range(4))
    DB = SBLK
    ND = idxb.shape[0]
    NPB = (ND + 15) // 16

    @functools.partial(
        pl.kernel,
        out_type=out_t,
        mesh=_vmesh(),
        scratch_types=[pltpu.VMEM_SHARED((N, H), jnp.float32),
                       pltpu.VMEM((2 * DB, H), jnp.float32),
                       pltpu.VMEM((2, 1, SBLK), jnp.int32),
                       pltpu.SemaphoreType.DMA,
                       pltpu.SemaphoreType.DMA,
                       pltpu.SemaphoreType.DMA,
                       pltpu.SemaphoreType.DMA])
    def k(m0h, m1h, m2h, m3h, ih, zh, o0, o1, o2, o3,
          acc, mbuf, ibuf, si0, si1, sm0, sm1):
        c = jax.lax.axis_index("c")
        s = jax.lax.axis_index("s")
        r0 = s * NPS
        isems = (si0, si1)
        msems = (sm0, sm1)

        def do_chunk(mh, oh):
            @pl.when(s < 15)
            def _():
                pltpu.sync_copy(zh, acc.at[pl.ds(r0, NPS)])

            @pl.when(s == 15)
            def _():
                pltpu.sync_copy(zh.at[pl.ds(0, N - 15 * NPS)],
                                acc.at[pl.ds(15 * NPS, N - 15 * NPS)])

            plsc.subcore_barrier()

            def issue(b, par):
                blk = b * 16 + s

                @pl.when(blk < ND)
                def _():
                    pltpu.make_async_copy(
                        ih.at[pl.ds(blk, 1)], ibuf.at[par],
                        isems[par]).start()
                    pltpu.make_async_copy(
                        mh.at[pl.ds(blk * DB, DB)],
                        mbuf.at[pl.ds(par * DB, DB)], msems[par]).start()

            def consume(b, par):
                blk = b * 16 + s

                @pl.when(blk < ND)
                def _():
                    pltpu.make_async_copy(
                        ih.at[pl.ds(blk, 1)], ibuf.at[par],
                        isems[par]).wait()
                    pltpu.make_async_copy(
                        mh.at[pl.ds(blk * DB, DB)],
                        mbuf.at[pl.ds(par * DB, DB)], msems[par]).wait()
                    pltpu.sync_copy(
                        mbuf.at[pl.ds(par * DB, DB)],
                        acc.at[ibuf.at[par, 0]], add=True)

            issue(0, 0)

            @pl.loop(0, (NPB + 1) // 2)
            def _(i):
                b0 = 2 * i
                issue(b0 + 1, 1)
                consume(b0, 0)
                issue(b0 + 2, 0)
                consume(b0 + 1, 1)

            plsc.subcore_barrier()

            @pl.when(s < 15)
            def _():
                pltpu.sync_copy(acc.at[pl.ds(r0, NPS)], oh.at[pl.ds(r0, NPS)])

            @pl.when(s == 15)
            def _():
                pltpu.sync_copy(acc.at[pl.ds(15 * NPS, N - 15 * NPS)],
                                oh.at[pl.ds(15 * NPS, N - 15 * NPS)])

            plsc.subcore_barrier()

        @pl.when(c == 0)
        def _():
            do_chunk(m0h, o0)
            do_chunk(m1h, o1)

        @pl.when(c == 1)
        def _():
            do_chunk(m2h, o2)
            do_chunk(m3h, o3)

    return k(m0, m1, m2, m3, idxb, zrows)


def _nodeupd_body(x_ref, v_ref, *rest):
    aggs = rest[:8]
    sw1_ref, sw2_ref, vm_ref, cst_ref, xo_ref, vo_ref = rest[8:]
    sb1 = cst_ref[0:1, :]
    sb2 = cst_ref[1:2, :]
    x = x_ref[...]
    summed = []
    for f in range(4):
        t = aggs[f][...]
        for h in range(1, 2):
            t = t + aggs[4 * h + f][...]
        summed.append(t)
    xu = jnp.concatenate([x, summed[0]], axis=1)
    t = jnp.dot(xu, sw1_ref[...], preferred_element_type=jnp.float32) + sb1
    t = t * jax.nn.sigmoid(t)
    xo_ref[...] = x + jnp.dot(t, sw2_ref[...],
                              preferred_element_type=jnp.float32) + sb2
    v = v_ref[...]
    comps = []
    for i in range(3):
        vi = v[:, i * H:(i + 1) * H]
        comps.append(vi + summed[1 + i] +
                     jnp.dot(vi, vm_ref[...],
                             preferred_element_type=jnp.float32))
    vo_ref[...] = jnp.concatenate(comps, axis=1)


def _nodeupd(x, v_flat, aggsets, sw1, sw2, vm, cst):
    flat = [a for s in aggsets for a in s]
    return pl.pallas_call(
        _nodeupd_body,
        grid=(N // BN,),
        in_specs=[pl.BlockSpec((BN, H), lambda i: (i, 0)),
                  pl.BlockSpec((BN, 3 * H), lambda i: (i, 0))] +
                 [pl.BlockSpec((BN, H), lambda i: (i, 0))] * 8 +
                 [pl.BlockSpec((2 * H, H), lambda i: (0, 0)),
                  pl.BlockSpec((H, H), lambda i: (0, 0)),
                  pl.BlockSpec((H, H), lambda i: (0, 0)),
                  pl.BlockSpec((8, H), lambda i: (0, 0))],
        out_specs=[pl.BlockSpec((BN, H), lambda i: (i, 0)),
                   pl.BlockSpec((BN, 3 * H), lambda i: (i, 0))],
        out_shape=[jax.ShapeDtypeStruct((N, H), jnp.float32),
                   jax.ShapeDtypeStruct((N, 3 * H), jnp.float32)],
    )(x, v_flat, *flat, sw1, sw2, vm, cst)


def _readout_body(v_ref, bf_ref, wf_ref, dip_ref, mol_ref):
    k = pl.program_id(0)

    @pl.when(k == 0)
    def _():
        mol_ref[...] = jnp.zeros_like(mol_ref)

    seg = jax.lax.broadcasted_iota(jnp.int32, (1, NGRAPH), 1).astype(jnp.float32)
    oh = (bf_ref[...] == seg).astype(jnp.float32)
    mol_ref[...] += jax.lax.dot_general(
        oh, v_ref[...], (((0,), (0,)), ((), ())),
        preferred_element_type=jnp.float32)

    @pl.when(k == N // BN - 1)
    def _():
        mol = mol_ref[...]
        wf = wf_ref[0:1, :]
        lane = jax.lax.broadcasted_iota(jnp.int32, (1, 128), 1)
        acc = jnp.zeros((NGRAPH, 128), jnp.float32)
        for i in range(3):
            ti = jnp.sum(mol[:, i * H:(i + 1) * H] * wf, axis=1,
                         keepdims=True)
            acc += ti * (lane == i).astype(jnp.float32)
        dip_ref[...] = acc


def _readout(v_flat, bf, wfr):
    return pl.pallas_call(
        _readout_body,
        grid=(N // BN,),
        in_specs=[pl.BlockSpec((BN, 3 * H), lambda i: (i, 0)),
                  pl.BlockSpec((BN, 1), lambda i: (i, 0)),
                  pl.BlockSpec((8, H), lambda i: (0, 0))],
        out_specs=pl.BlockSpec((NGRAPH, 128), lambda i: (0, 0)),
        out_shape=jax.ShapeDtypeStruct((NGRAPH, 128), jnp.float32),
        scratch_shapes=[pltpu.VMEM((NGRAPH, 3 * H), jnp.float32)],
    )(v_flat, bf, wfr)


def kernel(params, pos, z, edge_index, batch):
    row = edge_index[0].astype(jnp.int32)
    col = edge_index[1].astype(jnp.int32)
    NCH = 2
    EC = E // NCH
    rows2 = [row[h * EC:(h + 1) * EC].reshape(1, EC) for h in range(NCH)]
    cols2 = [col[h * EC:(h + 1) * EC].reshape(1, EC) for h in range(NCH)]
    rowbs = [row[h * EC:(h + 1) * EC].reshape(EC // SBLK, SBLK)
             for h in range(NCH)]

    emb_pad = jnp.zeros((128, H), jnp.float32).at[:VOCAB].set(params['emb'])
    zf = z.astype(jnp.float32).reshape(N, 1)
    x = _embed(zf, emb_pad)

    v_flat = jnp.broadcast_to(pos[:, :, None], (N, 3, H)).astype(
        jnp.float32).reshape(N, 3 * H)

    zrows = jnp.zeros((NPS, H), jnp.float32)
    outs = []
    for lp in params['layers']:
        wab = jnp.concatenate([lp['mw1'][:NFD], lp['mw1'][NFD:2 * NFD]],
                              axis=1)
        cst_e = jnp.zeros((8, NFD), jnp.float32)
        cst_e = cst_e.at[0].set(lp['mw1'][2 * NFD])
        cst_e = cst_e.at[1].set(lp['mb1'])
        cst_e = cst_e.at[2].set(lp['mb2'])
        cst_n = jnp.zeros((8, H), jnp.float32)
        cst_n = cst_n.at[0].set(lp['sb1'])
        cst_n = cst_n.at[1].set(lp['sb2'])

        a, b = _nodeproj(x, v_flat, wab)
        w2b = lp['mw2'].astype(jnp.bfloat16)
        aggs = []
        for h in range(NCH):
            ga = _sc_gather(a, rows2[h], GW)
            gb = _sc_gather(b, cols2[h], GW)
            m0, m1, m2, m3 = _edge_mlp(ga, gb, w2b, cst_e)
            aggs.append(_sc_scatter_add(m0, m1, m2, m3, rowbs[h], zrows))
        x, v_flat = _nodeupd(x, v_flat, aggs,
                             lp['sw1'], lp['sw2'], lp['vm'], cst_n)
        outs.append(x)
        outs.append(v_flat.reshape(N, 3, H))

    bf = batch.astype(jnp.float32).reshape(N, 1)
    wfr = jnp.zeros((8, H), jnp.float32).at[0].set(params['wf'][:, 0])
    dipfull = _readout(v_flat, bf, wfr)
    dip = dipfull[:NGRAPH, :3]
    return (dip, *outs)


# speedup vs baseline: 1.1998x; 1.0102x over previous
import functools

import jax
import jax.numpy as jnp
from jax.experimental import pallas as pl
from jax.experimental.pallas import tpu as pltpu
from jax.experimental.pallas import tpu_sc as plsc

N = 10000
E = 160000
H = 128
NFD = 4 * H
NGRAPH = 64
VOCAB = 100

BN = 1000
BE = 1600
GW = 128
SBLK = 128
NSB = E // SBLK
NPS = 640


_vmesh = functools.partial(
    plsc.VectorSubcoreMesh, core_axis_name="c", subcore_axis_name="s")


def _embed_body(zf_ref, emb_ref, x_ref):
    lane = jax.lax.broadcasted_iota(jnp.int32, (1, 128), 1).astype(jnp.float32)
    oh = (zf_ref[...] == lane).astype(jnp.float32)
    x_ref[...] = jnp.dot(oh, emb_ref[...],
                         preferred_element_type=jnp.float32)


def _embed(zf, emb_pad):
    return pl.pallas_call(
        _embed_body,
        grid=(N // BN,),
        in_specs=[pl.BlockSpec((BN, 1), lambda i: (i, 0)),
                  pl.BlockSpec((128, 128), lambda i: (0, 0))],
        out_specs=pl.BlockSpec((BN, H), lambda i: (i, 0)),
        out_shape=jax.ShapeDtypeStruct((N, H), jnp.float32),
    )(zf, emb_pad)


def _bf16_bits(x):
    u = jax.lax.bitcast_convert_type(x, jnp.int32)
    r = u + 0x7FFF + jnp.bitwise_and(jnp.right_shift(u, 16), 1)
    return jnp.bitwise_and(jnp.right_shift(r, 16), 0xFFFF)


def _nodeproj_body(x_ref, v_ref, wab_ref, a_ref, b_ref):
    nf = jnp.concatenate([x_ref[...], v_ref[...]], axis=1)
    ab = jnp.dot(nf, wab_ref[...], preferred_element_type=jnp.float32)
    v = v_ref[...]
    comps = [jnp.mean(v[:, i * H:(i + 1) * H], axis=1, keepdims=True)
             for i in range(3)]
    ppw = _bf16_bits(jnp.concatenate(
        comps + [jnp.zeros((v.shape[0], 125), jnp.float32)], axis=1))
    ha = NFD // 2
    aw = jnp.bitwise_or(_bf16_bits(ab[:, 0 * ha:1 * ha]),
                        jnp.left_shift(_bf16_bits(ab[:, 1 * ha:2 * ha]), 16))
    bw = jnp.bitwise_or(_bf16_bits(ab[:, 2 * ha:3 * ha]),
                        jnp.left_shift(_bf16_bits(ab[:, 3 * ha:4 * ha]), 16))
    a_ref[...] = jnp.concatenate([aw, ppw], axis=1)
    b_ref[...] = jnp.concatenate([bw, ppw], axis=1)


def _nodeproj(x, v_flat, wab):
    return pl.pallas_call(
        _nodeproj_body,
        grid=(N // BN,),
        in_specs=[pl.BlockSpec((BN, H), lambda i: (i, 0)),
                  pl.BlockSpec((BN, 3 * H), lambda i: (i, 0)),
                  pl.BlockSpec((NFD, 2 * NFD), lambda i: (0, 0))],
        out_specs=[pl.BlockSpec((BN, 384), lambda i: (i, 0))] * 2,
        out_shape=[jax.ShapeDtypeStruct((N, 384), jnp.int32)] * 2,
    )(x, v_flat, wab)


def _sc_gather(table, idx2, window):
    d = table.shape[1]
    e = idx2.shape[1]

    @functools.partial(
        pl.kernel,
        out_type=jax.ShapeDtypeStruct((e, d), table.dtype),
        mesh=_vmesh())
    def k(tab_hbm, i_hbm, o_hbm):
        def body(i_vmem, o_vmem):
            pltpu.sync_copy(tab_hbm.at[i_vmem.at[0]], o_vmem)

        pltpu.emit_pipeline(
            body,
            grid=(e // window,),
            in_specs=[pl.BlockSpec((1, window), lambda i: (0, i))],
            out_specs=[pl.BlockSpec((window, d), lambda i: (i, 0))],
            core_axis_name=("c", "s"),
            dimension_semantics=(pltpu.PARALLEL,),
        )(i_hbm, o_hbm)

    return k(table, idx2)


def _sc_gather_ab(ta, tb, ridx, cidx):
    d = ta.shape[1]
    e = ridx.shape[1]

    @functools.partial(
        pl.kernel,
        out_type=(jax.ShapeDtypeStruct((e, d), ta.dtype),
                  jax.ShapeDtypeStruct((e, d), tb.dtype)),
        mesh=_vmesh())
    def k(ta_hbm, tb_hbm, ri_hbm, ci_hbm, oa_hbm, ob_hbm):
        c = jax.lax.axis_index("c")

        def run(tab, ih, oh):
            def body(i_vmem, o_vmem):
                pltpu.sync_copy(tab.at[i_vmem.at[0]], o_vmem)

            pltpu.emit_pipeline(
                body,
                grid=(e // GW,),
                in_specs=[pl.BlockSpec((1, GW), lambda i: (0, i))],
                out_specs=[pl.BlockSpec((GW, d), lambda i: (i, 0))],
                core_axis_name="s",
                dimension_semantics=(pltpu.PARALLEL,),
            )(ih, oh)

        @pl.when(c == 0)
        def _():
            run(ta_hbm, ri_hbm, oa_hbm)

        @pl.when(c == 1)
        def _():
            run(tb_hbm, ci_hbm, ob_hbm)

    return k(ta, tb, ridx, cidx)


def _unpack_lo(w):
    return jax.lax.bitcast_convert_type(jnp.left_shift(w, 16), jnp.float32)


def _unpack_hi(w):
    return jax.lax.bitcast_convert_type(
        jnp.bitwise_and(w, jnp.int32(-65536)), jnp.float32)


def _edge_body(ga_ref, gb_ref, w2_ref,
               cst_ref, m0_ref, m1_ref, m2_ref, m3_ref):
    wd = cst_ref[0:1, :]
    b1 = cst_ref[1:2, :]
    b2 = cst_ref[2:3, :]
    ha = NFD // 2
    gaw = ga_ref[...]
    gbw = gb_ref[...]
    dr = _unpack_lo(gaw[:, ha:]) - _unpack_lo(gbw[:, ha:])
    dist = jnp.sqrt(jnp.sum(dr * dr, axis=1, keepdims=True))
    ga = jnp.concatenate([_unpack_lo(gaw[:, :ha]), _unpack_hi(gaw[:, :ha])],
                         axis=1)
    gb = jnp.concatenate([_unpack_lo(gbw[:, :ha]), _unpack_hi(gbw[:, :ha])],
                         axis=1)
    u = ga + gb + dist * wd + b1
    h1 = u * jax.nn.sigmoid(u)
    m = jnp.dot(h1.astype(jnp.bfloat16), w2_ref[...],
                preferred_element_type=jnp.float32) + b2
    m = m * jax.nn.sigmoid(m)
    m0_ref[...] = m[:, 0 * H:1 * H]
    m1_ref[...] = m[:, 1 * H:2 * H]
    m2_ref[...] = m[:, 2 * H:3 * H]
    m3_ref[...] = m[:, 3 * H:4 * H]


def _edge_mlp(ga, gb, w2, cst):
    e = ga.shape[0]
    outs = [jax.ShapeDtypeStruct((e, H), jnp.float32)] * 4
    return pl.pallas_call(
        _edge_body,
        grid=(e // BE,),
        in_specs=[pl.BlockSpec((BE, 384), lambda i: (i, 0)),
                  pl.BlockSpec((BE, 384), lambda i: (i, 0)),
                  pl.BlockSpec((NFD, NFD), lambda i: (0, 0)),
                  pl.BlockSpec((8, NFD), lambda i: (0, 0))],
        out_specs=[pl.BlockSpec((BE, H), lambda i: (i, 0))] * 4,
        out_shape=outs,
    )(ga, gb, w2, cst)


def _sc_scatter_add(m0, m1, m2, m3, idxb, zrows):
    out_t = tuple(jax.ShapeDtypeStruct((N, H), jnp.float32) for _ in range(4))
    DB = SBLK
    ND = idxb.shape[0]
    NPB = (ND + 15) // 16

    @functools.partial(
        pl.kernel,
        out_type=out_t,
        mesh=_vmesh(),
        scratch_types=[pltpu.VMEM_SHARED((N, H), jnp.float32),
                       pltpu.VMEM((2 * DB, H), jnp.float32),
                       pltpu.VMEM((2, 1, SBLK), jnp.int32),
                       pltpu.SemaphoreType.DMA,
                       pltpu.SemaphoreType.DMA,
                       pltpu.SemaphoreType.DMA,
                       pltpu.SemaphoreType.DMA])
    def k(m0h, m1h, m2h, m3h, ih, zh, o0, o1, o2, o3,
          acc, mbuf, ibuf, si0, si1, sm0, sm1):
        c = jax.lax.axis_index("c")
        s = jax.lax.axis_index("s")
        r0 = s * NPS
        isems = (si0, si1)
        msems = (sm0, sm1)

        def do_chunk(mh, oh):
            @pl.when(s < 15)
            def _():
                pltpu.sync_copy(zh, acc.at[pl.ds(r0, NPS)])

            @pl.when(s == 15)
            def _():
                pltpu.sync_copy(zh.at[pl.ds(0, N - 15 * NPS)],
                                acc.at[pl.ds(15 * NPS, N - 15 * NPS)])

            plsc.subcore_barrier()

            def issue(b, par):
                blk = b * 16 + s

                @pl.when(blk < ND)
                def _():
                    pltpu.make_async_copy(
                        ih.at[pl.ds(blk, 1)], ibuf.at[par],
                        isems[par]).start()
                    pltpu.make_async_copy(
                        mh.at[pl.ds(blk * DB, DB)],
                        mbuf.at[pl.ds(par * DB, DB)], msems[par]).start()

            def consume(b, par):
                blk = b * 16 + s

                @pl.when(blk < ND)
                def _():
                    pltpu.make_async_copy(
                        ih.at[pl.ds(blk, 1)], ibuf.at[par],
                        isems[par]).wait()
                    pltpu.make_async_copy(
                        mh.at[pl.ds(blk * DB, DB)],
                        mbuf.at[pl.ds(par * DB, DB)], msems[par]).wait()
                    pltpu.sync_copy(
                        mbuf.at[pl.ds(par * DB, DB)],
                        acc.at[ibuf.at[par, 0]], add=True)

            issue(0, 0)

            @pl.loop(0, (NPB + 1) // 2)
            def _(i):
                b0 = 2 * i
                issue(b0 + 1, 1)
                consume(b0, 0)
                issue(b0 + 2, 0)
                consume(b0 + 1, 1)

            plsc.subcore_barrier()

            @pl.when(s < 15)
            def _():
                pltpu.sync_copy(acc.at[pl.ds(r0, NPS)], oh.at[pl.ds(r0, NPS)])

            @pl.when(s == 15)
            def _():
                pltpu.sync_copy(acc.at[pl.ds(15 * NPS, N - 15 * NPS)],
                                oh.at[pl.ds(15 * NPS, N - 15 * NPS)])

            plsc.subcore_barrier()

        @pl.when(c == 0)
        def _():
            do_chunk(m0h, o0)
            do_chunk(m1h, o1)

        @pl.when(c == 1)
        def _():
            do_chunk(m2h, o2)
            do_chunk(m3h, o3)

    return k(m0, m1, m2, m3, idxb, zrows)


def _nodeupd_body(x_ref, v_ref, *rest):
    aggs = rest[:8]
    sw1_ref, sw2_ref, vm_ref, cst_ref, xo_ref, vo_ref = rest[8:]
    sb1 = cst_ref[0:1, :]
    sb2 = cst_ref[1:2, :]
    x = x_ref[...]
    summed = []
    for f in range(4):
        t = aggs[f][...]
        for h in range(1, 2):
            t = t + aggs[4 * h + f][...]
        summed.append(t)
    xu = jnp.concatenate([x, summed[0]], axis=1)
    t = jnp.dot(xu, sw1_ref[...], preferred_element_type=jnp.float32) + sb1
    t = t * jax.nn.sigmoid(t)
    xo_ref[...] = x + jnp.dot(t, sw2_ref[...],
                              preferred_element_type=jnp.float32) + sb2
    v = v_ref[...]
    comps = []
    for i in range(3):
        vi = v[:, i * H:(i + 1) * H]
        comps.append(vi + summed[1 + i] +
                     jnp.dot(vi, vm_ref[...],
                             preferred_element_type=jnp.float32))
    vo_ref[...] = jnp.concatenate(comps, axis=1)


def _nodeupd(x, v_flat, aggsets, sw1, sw2, vm, cst):
    flat = [a for s in aggsets for a in s]
    return pl.pallas_call(
        _nodeupd_body,
        grid=(N // BN,),
        in_specs=[pl.BlockSpec((BN, H), lambda i: (i, 0)),
                  pl.BlockSpec((BN, 3 * H), lambda i: (i, 0))] +
                 [pl.BlockSpec((BN, H), lambda i: (i, 0))] * 8 +
                 [pl.BlockSpec((2 * H, H), lambda i: (0, 0)),
                  pl.BlockSpec((H, H), lambda i: (0, 0)),
                  pl.BlockSpec((H, H), lambda i: (0, 0)),
                  pl.BlockSpec((8, H), lambda i: (0, 0))],
        out_specs=[pl.BlockSpec((BN, H), lambda i: (i, 0)),
                   pl.BlockSpec((BN, 3 * H), lambda i: (i, 0))],
        out_shape=[jax.ShapeDtypeStruct((N, H), jnp.float32),
                   jax.ShapeDtypeStruct((N, 3 * H), jnp.float32)],
    )(x, v_flat, *flat, sw1, sw2, vm, cst)


def _readout_body(v_ref, bf_ref, wf_ref, dip_ref, mol_ref):
    k = pl.program_id(0)

    @pl.when(k == 0)
    def _():
        mol_ref[...] = jnp.zeros_like(mol_ref)

    seg = jax.lax.broadcasted_iota(jnp.int32, (1, NGRAPH), 1).astype(jnp.float32)
    oh = (bf_ref[...] == seg).astype(jnp.float32)
    mol_ref[...] += jax.lax.dot_general(
        oh, v_ref[...], (((0,), (0,)), ((), ())),
        preferred_element_type=jnp.float32)

    @pl.when(k == N // BN - 1)
    def _():
        mol = mol_ref[...]
        wf = wf_ref[0:1, :]
        lane = jax.lax.broadcasted_iota(jnp.int32, (1, 128), 1)
        acc = jnp.zeros((NGRAPH, 128), jnp.float32)
        for i in range(3):
            ti = jnp.sum(mol[:, i * H:(i + 1) * H] * wf, axis=1,
                         keepdims=True)
            acc += ti * (lane == i).astype(jnp.float32)
        dip_ref[...] = acc


def _readout(v_flat, bf, wfr):
    return pl.pallas_call(
        _readout_body,
        grid=(N // BN,),
        in_specs=[pl.BlockSpec((BN, 3 * H), lambda i: (i, 0)),
                  pl.BlockSpec((BN, 1), lambda i: (i, 0)),
                  pl.BlockSpec((8, H), lambda i: (0, 0))],
        out_specs=pl.BlockSpec((NGRAPH, 128), lambda i: (0, 0)),
        out_shape=jax.ShapeDtypeStruct((NGRAPH, 128), jnp.float32),
        scratch_shapes=[pltpu.VMEM((NGRAPH, 3 * H), jnp.float32)],
    )(v_flat, bf, wfr)


def kernel(params, pos, z, edge_index, batch):
    row = edge_index[0].astype(jnp.int32)
    col = edge_index[1].astype(jnp.int32)
    NCH = 2
    EC = E // NCH
    rows2 = [row[h * EC:(h + 1) * EC].reshape(1, EC) for h in range(NCH)]
    cols2 = [col[h * EC:(h + 1) * EC].reshape(1, EC) for h in range(NCH)]
    rowbs = [row[h * EC:(h + 1) * EC].reshape(EC // SBLK, SBLK)
             for h in range(NCH)]

    emb_pad = jnp.zeros((128, H), jnp.float32).at[:VOCAB].set(params['emb'])
    zf = z.astype(jnp.float32).reshape(N, 1)
    x = _embed(zf, emb_pad)

    v_flat = jnp.broadcast_to(pos[:, :, None], (N, 3, H)).astype(
        jnp.float32).reshape(N, 3 * H)

    zrows = jnp.zeros((NPS, H), jnp.float32)
    outs = []
    for lp in params['layers']:
        wab = jnp.concatenate([lp['mw1'][:NFD], lp['mw1'][NFD:2 * NFD]],
                              axis=1)
        cst_e = jnp.zeros((8, NFD), jnp.float32)
        cst_e = cst_e.at[0].set(lp['mw1'][2 * NFD])
        cst_e = cst_e.at[1].set(lp['mb1'])
        cst_e = cst_e.at[2].set(lp['mb2'])
        cst_n = jnp.zeros((8, H), jnp.float32)
        cst_n = cst_n.at[0].set(lp['sb1'])
        cst_n = cst_n.at[1].set(lp['sb2'])

        a, b = _nodeproj(x, v_flat, wab)
        w2b = lp['mw2'].astype(jnp.bfloat16)
        aggs = []
        for h in range(NCH):
            ga, gb = _sc_gather_ab(a, b, rows2[h], cols2[h])
            m0, m1, m2, m3 = _edge_mlp(ga, gb, w2b, cst_e)
            aggs.append(_sc_scatter_add(m0, m1, m2, m3, rowbs[h], zrows))
        x, v_flat = _nodeupd(x, v_flat, aggs,
                             lp['sw1'], lp['sw2'], lp['vm'], cst_n)
        outs.append(x)
        outs.append(v_flat.reshape(N, 3, H))

    bf = batch.astype(jnp.float32).reshape(N, 1)
    wfr = jnp.zeros((8, H), jnp.float32).at[0].set(params['wf'][:, 0])
    dipfull = _readout(v_flat, bf, wfr)
    dip = dipfull[:NGRAPH, :3]
    return (dip, *outs)
